# baseline (speedup 1.0000x reference)
import functools
import math

import jax
import jax.numpy as jnp
from jax.experimental import pallas as pl
from jax.experimental.pallas import tpu as pltpu

_NUM_LANDMARKS = 64
_HALF = 32
_CQ = 64
_KW = _CQ + _HALF
_NEG = float("-inf")


def _ssa_block_kernel(q_ref, k_ref, v_ref, o_ref,
                      band_ref, lmm_ref, npad_ref, lmk_ref, lmv_ref,
                      *, bq, stride, nblk):
    h = pl.program_id(0)
    i = pl.program_id(1)
    d = q_ref.shape[-1]
    scale = 1.0 / math.sqrt(d)

    @pl.when((h == 0) & (i == 0))
    def _():
        r = jax.lax.broadcasted_iota(jnp.int32, (_CQ, _KW), 0)
        c = jax.lax.broadcasted_iota(jnp.int32, (_CQ, _KW), 1)
        band_ref[0:_CQ, :] = jnp.where((c >= r - _HALF) & (c <= r), 0.0, _NEG)
        band_ref[_CQ:, :] = jnp.where((c >= r) & (c <= r + _HALF), 0.0, _NEG)
        rl = jax.lax.broadcasted_iota(jnp.int32, (bq, _NUM_LANDMARKS), 0)
        ll = jax.lax.broadcasted_iota(jnp.int32, (bq, _NUM_LANDMARKS), 1)
        for blk in range(nblk):
            lmm_ref[blk * bq:(blk + 1) * bq, :] = jnp.where(
                ll * stride > blk * bq + rl, _NEG, 0.0)
        rp = jax.lax.broadcasted_iota(jnp.int32, (_CQ, 1), 0)
        npad_ref[...] = jnp.maximum(_HALF - rp, 0).astype(jnp.float32)

    @pl.when(i == 0)
    def _():
        lmk_ref[...] = k_ref[0].reshape(_NUM_LANDMARKS, stride, d)[:, 0, :]
        lmv_ref[...] = v_ref[0].reshape(_NUM_LANDMARKS, stride, d)[:, 0, :]

    q = q_ref[0] * scale

    lm_scores = jax.lax.dot_general(
        q, lmk_ref[...], (((1,), (1,)), ((), ())),
        preferred_element_type=jnp.float32,
    ) + lmm_ref[pl.ds(i * bq, bq), :]
    m2 = jnp.max(lm_scores, axis=1, keepdims=True)
    e2 = jnp.exp(lm_scores - m2)
    den2 = jnp.sum(e2, axis=1, keepdims=True)
    glob = jax.lax.dot_general(
        e2, lmv_ref[...], (((1,), (0,)), ((), ())),
        preferred_element_type=jnp.float32,
    ) / den2

    for c0 in range(bq // _CQ):
        qc = q[c0 * _CQ:(c0 + 1) * _CQ, :]
        if c0 == 0:
            is_edge = i == 0
            kstart = jnp.maximum(i * bq - _HALF, 0)
            band = band_ref[pl.ds(jnp.where(is_edge, 0, _CQ), _CQ), :]
            edge01 = jnp.where(is_edge, 1.0, 0.0)
            npad = npad_ref[...] * edge01
        else:
            kstart = i * bq + c0 * _CQ - _HALF
            band = band_ref[_CQ:, :]
            npad = None
        kc = k_ref[0, pl.ds(kstart, _KW), :]
        vc = v_ref[0, pl.ds(kstart, _KW), :]

        sc = jax.lax.dot_general(
            qc, kc, (((1,), (1,)), ((), ())),
            preferred_element_type=jnp.float32,
        ) + band
        m = jnp.max(sc, axis=1, keepdims=True)
        if npad is not None:
            m = jnp.where(npad > 0, jnp.maximum(m, 0.0), m)
        e = jnp.exp(sc - m)
        den = jnp.sum(e, axis=1, keepdims=True)
        if npad is not None:
            den = den + npad * jnp.exp(-m)
        loc = jax.lax.dot_general(
            e, vc, (((1,), (0,)), ((), ())),
            preferred_element_type=jnp.float32,
        ) / den
        o_ref[0, c0 * _CQ:(c0 + 1) * _CQ, :] = (
            loc + glob[c0 * _CQ:(c0 + 1) * _CQ, :]
        ).astype(o_ref.dtype)


@jax.jit
def kernel(query, key, value):
    b, h, s, d = query.shape
    assert b == 1
    bq = 256
    nblk = s // bq
    stride = s // _NUM_LANDMARKS

    grid = (h, nblk)
    out = pl.pallas_call(
        functools.partial(_ssa_block_kernel, bq=bq, stride=stride, nblk=nblk),
        grid=grid,
        in_specs=[
            pl.BlockSpec((1, bq, d), lambda hh, ii: (hh, ii, 0)),
            pl.BlockSpec((1, s, d), lambda hh, ii: (hh, 0, 0)),
            pl.BlockSpec((1, s, d), lambda hh, ii: (hh, 0, 0)),
        ],
        out_specs=pl.BlockSpec((1, bq, d), lambda hh, ii: (hh, ii, 0)),
        out_shape=jax.ShapeDtypeStruct((h, s, d), query.dtype),
        scratch_shapes=[
            pltpu.VMEM((2 * _CQ, _KW), jnp.float32),
            pltpu.VMEM((s, _NUM_LANDMARKS), jnp.float32),
            pltpu.VMEM((_CQ, 1), jnp.float32),
            pltpu.VMEM((_NUM_LANDMARKS, d), jnp.float32),
            pltpu.VMEM((_NUM_LANDMARKS, d), jnp.float32),
        ],
    )(query[0], key[0], value[0])
    return out[None]


# speedup vs baseline: 1.0757x; 1.0757x over previous
import functools
import math

import jax
import jax.numpy as jnp
from jax.experimental import pallas as pl
from jax.experimental.pallas import tpu as pltpu

_NUM_LANDMARKS = 64
_HALF = 32
_CQ = 64
_KW = _CQ + _HALF
_NEG = float("-inf")


def _ssa_block_kernel(q_ref, k_ref, v_ref, o_ref,
                      band_ref, lmm_ref, npad_ref, lmk_ref, lmv_ref,
                      *, bq, stride, nblk):
    h = pl.program_id(0)
    i = pl.program_id(1)
    d = q_ref.shape[-1]
    scale = 1.0 / math.sqrt(d)
    q_ref = q_ref.at[0]
    k_ref = k_ref.at[0]
    v_ref = v_ref.at[0]
    o_ref = o_ref.at[0]

    @pl.when((h == 0) & (i == 0))
    def _():
        r = jax.lax.broadcasted_iota(jnp.int32, (_CQ, _KW), 0)
        c = jax.lax.broadcasted_iota(jnp.int32, (_CQ, _KW), 1)
        band_ref[0:_CQ, :] = jnp.where((c >= r - _HALF) & (c <= r), 0.0, _NEG)
        band_ref[_CQ:, :] = jnp.where((c >= r) & (c <= r + _HALF), 0.0, _NEG)
        rl = jax.lax.broadcasted_iota(jnp.int32, (bq, _NUM_LANDMARKS), 0)
        ll = jax.lax.broadcasted_iota(jnp.int32, (bq, _NUM_LANDMARKS), 1)
        for blk in range(nblk):
            lmm_ref[blk * bq:(blk + 1) * bq, :] = jnp.where(
                ll * stride > blk * bq + rl, _NEG, 0.0)
        rp = jax.lax.broadcasted_iota(jnp.int32, (_CQ, 1), 0)
        npad_ref[...] = jnp.maximum(_HALF - rp, 0).astype(jnp.float32)

    @pl.when(i == 0)
    def _():
        lmk_ref[...] = k_ref[0].reshape(_NUM_LANDMARKS, stride, d)[:, 0, :]
        lmv_ref[...] = v_ref[0].reshape(_NUM_LANDMARKS, stride, d)[:, 0, :]

    q = q_ref[0] * scale

    lm_scores = jax.lax.dot_general(
        q, lmk_ref[...], (((1,), (1,)), ((), ())),
        preferred_element_type=jnp.float32,
    ) + lmm_ref[pl.ds(i * bq, bq), :]
    m2 = jnp.max(lm_scores, axis=1, keepdims=True)
    e2 = jnp.exp(lm_scores - m2)
    den2 = jnp.sum(e2, axis=1, keepdims=True)
    glob = jax.lax.dot_general(
        e2, lmv_ref[...], (((1,), (0,)), ((), ())),
        preferred_element_type=jnp.float32,
    ) / den2

    for c0 in range(bq // _CQ):
        qc = q[c0 * _CQ:(c0 + 1) * _CQ, :]
        if c0 == 0:
            is_edge = i == 0
            kstart = jnp.maximum(i * bq - _HALF, 0)
            band = band_ref[pl.ds(jnp.where(is_edge, 0, _CQ), _CQ), :]
            edge01 = jnp.where(is_edge, 1.0, 0.0)
            npad = npad_ref[...] * edge01
        else:
            kstart = i * bq + c0 * _CQ - _HALF
            band = band_ref[_CQ:, :]
            npad = None
        kc = k_ref[0, pl.ds(kstart, _KW), :]
        vc = v_ref[0, pl.ds(kstart, _KW), :]

        sc = jax.lax.dot_general(
            qc, kc, (((1,), (1,)), ((), ())),
            preferred_element_type=jnp.float32,
        ) + band
        m = jnp.max(sc, axis=1, keepdims=True)
        if npad is not None:
            m = jnp.where(npad > 0, jnp.maximum(m, 0.0), m)
        e = jnp.exp(sc - m)
        den = jnp.sum(e, axis=1, keepdims=True)
        if npad is not None:
            den = den + npad * jnp.exp(-m)
        loc = jax.lax.dot_general(
            e, vc, (((1,), (0,)), ((), ())),
            preferred_element_type=jnp.float32,
        ) / den
        o_ref[0, c0 * _CQ:(c0 + 1) * _CQ, :] = (
            loc + glob[c0 * _CQ:(c0 + 1) * _CQ, :]
        ).astype(o_ref.dtype)


@jax.jit
def kernel(query, key, value):
    b, h, s, d = query.shape
    assert b == 1
    bq = 256
    nblk = s // bq
    stride = s // _NUM_LANDMARKS

    grid = (h, nblk)
    out = pl.pallas_call(
        functools.partial(_ssa_block_kernel, bq=bq, stride=stride, nblk=nblk),
        grid=grid,
        in_specs=[
            pl.BlockSpec((1, 1, bq, d), lambda hh, ii: (0, hh, ii, 0)),
            pl.BlockSpec((1, 1, s, d), lambda hh, ii: (0, hh, 0, 0)),
            pl.BlockSpec((1, 1, s, d), lambda hh, ii: (0, hh, 0, 0)),
        ],
        out_specs=pl.BlockSpec((1, 1, bq, d), lambda hh, ii: (0, hh, ii, 0)),
        out_shape=jax.ShapeDtypeStruct((b, h, s, d), query.dtype),
        scratch_shapes=[
            pltpu.VMEM((2 * _CQ, _KW), jnp.float32),
            pltpu.VMEM((s, _NUM_LANDMARKS), jnp.float32),
            pltpu.VMEM((_CQ, 1), jnp.float32),
            pltpu.VMEM((_NUM_LANDMARKS, d), jnp.float32),
            pltpu.VMEM((_NUM_LANDMARKS, d), jnp.float32),
        ],
    )(query, key, value)
    return out


# speedup vs baseline: 1.2970x; 1.2057x over previous
import functools
import math

import jax
import jax.numpy as jnp
from jax.experimental import pallas as pl
from jax.experimental.pallas import tpu as pltpu

_NUM_LANDMARKS = 64
_HALF = 32
_CQ = 64
_KW = _CQ + _HALF
_NEG = float("-inf")


def _ssa_head_kernel(q_ref, k_ref, v_ref, o_ref,
                     band_ref, lmm_ref, npad_ref, lmk_ref, lmv_ref,
                     *, s, stride):
    h = pl.program_id(0)
    d = q_ref.shape[-1]
    scale = 1.0 / math.sqrt(d)

    @pl.when(h == 0)
    def _():
        r = jax.lax.broadcasted_iota(jnp.int32, (_CQ, _KW), 0)
        c = jax.lax.broadcasted_iota(jnp.int32, (_CQ, _KW), 1)
        band_ref[0:_CQ, :] = jnp.where((c >= r - _HALF) & (c <= r), 0.0, _NEG)
        band_ref[_CQ:, :] = jnp.where((c >= r) & (c <= r + _HALF), 0.0, _NEG)
        rl = jax.lax.broadcasted_iota(jnp.int32, (s, _NUM_LANDMARKS), 0)
        ll = jax.lax.broadcasted_iota(jnp.int32, (s, _NUM_LANDMARKS), 1)
        lmm_ref[...] = jnp.where(ll * stride > rl, _NEG, 0.0)
        rp = jax.lax.broadcasted_iota(jnp.int32, (_CQ, 1), 0)
        npad_ref[...] = jnp.maximum(_HALF - rp, 0).astype(jnp.float32)

    lmk_ref[...] = k_ref[0, 0].reshape(_NUM_LANDMARKS, stride, d)[:, 0, :]
    lmv_ref[...] = v_ref[0, 0].reshape(_NUM_LANDMARKS, stride, d)[:, 0, :]

    q = q_ref[0, 0] * scale

    lm_scores = jax.lax.dot_general(
        q, lmk_ref[...], (((1,), (1,)), ((), ())),
        preferred_element_type=jnp.float32,
    ) + lmm_ref[...]
    m2 = jnp.max(lm_scores, axis=1, keepdims=True)
    e2 = jnp.exp(lm_scores - m2)
    den2 = jnp.sum(e2, axis=1, keepdims=True)
    glob = jax.lax.dot_general(
        e2, lmv_ref[...], (((1,), (0,)), ((), ())),
        preferred_element_type=jnp.float32,
    ) / den2

    for c0 in range(s // _CQ):
        qc = q[c0 * _CQ:(c0 + 1) * _CQ, :]
        if c0 == 0:
            kstart = 0
            band = band_ref[0:_CQ, :]
        else:
            kstart = c0 * _CQ - _HALF
            band = band_ref[_CQ:, :]
        kc = k_ref[0, 0, kstart:kstart + _KW, :]
        vc = v_ref[0, 0, kstart:kstart + _KW, :]

        sc = jax.lax.dot_general(
            qc, kc, (((1,), (1,)), ((), ())),
            preferred_element_type=jnp.float32,
        ) + band
        m = jnp.max(sc, axis=1, keepdims=True)
        if c0 == 0:
            npad = npad_ref[...]
            m = jnp.where(npad > 0, jnp.maximum(m, 0.0), m)
        e = jnp.exp(sc - m)
        den = jnp.sum(e, axis=1, keepdims=True)
        if c0 == 0:
            den = den + npad * jnp.exp(-m)
        loc = jax.lax.dot_general(
            e, vc, (((1,), (0,)), ((), ())),
            preferred_element_type=jnp.float32,
        ) / den
        o_ref[0, 0, c0 * _CQ:(c0 + 1) * _CQ, :] = (
            loc + glob[c0 * _CQ:(c0 + 1) * _CQ, :]
        ).astype(o_ref.dtype)


@jax.jit
def kernel(query, key, value):
    b, h, s, d = query.shape
    assert b == 1
    stride = s // _NUM_LANDMARKS

    out = pl.pallas_call(
        functools.partial(_ssa_head_kernel, s=s, stride=stride),
        grid=(h,),
        in_specs=[
            pl.BlockSpec((1, 1, s, d), lambda hh: (0, hh, 0, 0)),
            pl.BlockSpec((1, 1, s, d), lambda hh: (0, hh, 0, 0)),
            pl.BlockSpec((1, 1, s, d), lambda hh: (0, hh, 0, 0)),
        ],
        out_specs=pl.BlockSpec((1, 1, s, d), lambda hh: (0, hh, 0, 0)),
        out_shape=jax.ShapeDtypeStruct((b, h, s, d), query.dtype),
        scratch_shapes=[
            pltpu.VMEM((2 * _CQ, _KW), jnp.float32),
            pltpu.VMEM((s, _NUM_LANDMARKS), jnp.float32),
            pltpu.VMEM((_CQ, 1), jnp.float32),
            pltpu.VMEM((_NUM_LANDMARKS, d), jnp.float32),
            pltpu.VMEM((_NUM_LANDMARKS, d), jnp.float32),
        ],
    )(query, key, value)
    return out


# speedup vs baseline: 2.5388x; 1.9574x over previous
import functools
import math

import jax
import jax.numpy as jnp
from jax.experimental import pallas as pl
from jax.experimental.pallas import tpu as pltpu

_NUM_LANDMARKS = 64
_HALF = 32
_CQ = 64
_KW = _CQ + _HALF
_NEG = float("-inf")


def _ssa_head_kernel(q_ref, k_ref, v_ref, o_ref,
                     band_ref, lmm_ref, kv0_ref, lmk_ref, lmv_ref, sc_ref,
                     *, s, stride):
    h = pl.program_id(0)
    d = q_ref.shape[-1]
    scale = 1.0 / math.sqrt(d)
    nchunk = s // _CQ

    @pl.when(h == 0)
    def _():
        r = jax.lax.broadcasted_iota(jnp.int32, (s, _KW), 0)
        c = jax.lax.broadcasted_iota(jnp.int32, (s, _KW), 1)
        rm = jax.lax.rem(r, _CQ)
        band_ref[...] = jnp.where((c >= rm) & (c <= rm + _HALF), 0.0, _NEG)
        rl = jax.lax.broadcasted_iota(jnp.int32, (s, _NUM_LANDMARKS), 0)
        ll = jax.lax.broadcasted_iota(jnp.int32, (s, _NUM_LANDMARKS), 1)
        lmm_ref[...] = jnp.where(ll * stride > rl, _NEG, 0.0)

    lmk_ref[...] = k_ref[0, 0].reshape(_NUM_LANDMARKS, stride, d)[:, 0, :]
    lmv_ref[...] = v_ref[0, 0].reshape(_NUM_LANDMARKS, stride, d)[:, 0, :]
    zeros = jnp.zeros((_HALF, d), jnp.float32)
    kv0_ref[0:_HALF, :] = zeros
    kv0_ref[_KW:_KW + _HALF, :] = zeros
    kv0_ref[_HALF:_KW, :] = k_ref[0, 0, 0:_CQ, :]
    kv0_ref[_KW + _HALF:, :] = v_ref[0, 0, 0:_CQ, :]

    q = q_ref[0, 0] * scale

    lm_scores = jax.lax.dot_general(
        q, lmk_ref[...], (((1,), (1,)), ((), ())),
        preferred_element_type=jnp.float32,
    ) + lmm_ref[...]
    m2 = jnp.max(lm_scores, axis=1, keepdims=True)
    e2 = jnp.exp(lm_scores - m2)
    w2 = e2 / jnp.sum(e2, axis=1, keepdims=True)
    glob = jax.lax.dot_general(
        w2, lmv_ref[...], (((1,), (0,)), ((), ())),
        preferred_element_type=jnp.float32,
    )

    for c0 in range(nchunk):
        qc = q[c0 * _CQ:(c0 + 1) * _CQ, :]
        if c0 == 0:
            kc = kv0_ref[0:_KW, :]
        else:
            kc = k_ref[0, 0, c0 * _CQ - _HALF:c0 * _CQ + _CQ, :]
        sc_ref[c0 * _CQ:(c0 + 1) * _CQ, :] = jax.lax.dot_general(
            qc, kc, (((1,), (1,)), ((), ())),
            preferred_element_type=jnp.float32,
        )

    scm = sc_ref[...] + band_ref[...]
    m = jnp.max(scm, axis=1, keepdims=True)
    e = jnp.exp(scm - m)
    sc_ref[...] = e / jnp.sum(e, axis=1, keepdims=True)

    for c0 in range(nchunk):
        wc = sc_ref[c0 * _CQ:(c0 + 1) * _CQ, :]
        if c0 == 0:
            vc = kv0_ref[_KW:, :]
        else:
            vc = v_ref[0, 0, c0 * _CQ - _HALF:c0 * _CQ + _CQ, :]
        loc = jax.lax.dot_general(
            wc, vc, (((1,), (0,)), ((), ())),
            preferred_element_type=jnp.float32,
        )
        o_ref[0, 0, c0 * _CQ:(c0 + 1) * _CQ, :] = (
            loc + glob[c0 * _CQ:(c0 + 1) * _CQ, :]
        ).astype(o_ref.dtype)


@jax.jit
def kernel(query, key, value):
    b, h, s, d = query.shape
    assert b == 1
    stride = s // _NUM_LANDMARKS

    out = pl.pallas_call(
        functools.partial(_ssa_head_kernel, s=s, stride=stride),
        grid=(h,),
        in_specs=[
            pl.BlockSpec((1, 1, s, d), lambda hh: (0, hh, 0, 0)),
            pl.BlockSpec((1, 1, s, d), lambda hh: (0, hh, 0, 0)),
            pl.BlockSpec((1, 1, s, d), lambda hh: (0, hh, 0, 0)),
        ],
        out_specs=pl.BlockSpec((1, 1, s, d), lambda hh: (0, hh, 0, 0)),
        out_shape=jax.ShapeDtypeStruct((b, h, s, d), query.dtype),
        scratch_shapes=[
            pltpu.VMEM((s, _KW), jnp.float32),
            pltpu.VMEM((s, _NUM_LANDMARKS), jnp.float32),
            pltpu.VMEM((2 * _KW, d), jnp.float32),
            pltpu.VMEM((_NUM_LANDMARKS, d), jnp.float32),
            pltpu.VMEM((_NUM_LANDMARKS, d), jnp.float32),
            pltpu.VMEM((s, _KW), jnp.float32),
        ],
    )(query, key, value)
    return out
